# Initial kernel scaffold; baseline (speedup 1.0000x reference)
import jax
import jax.numpy as jnp
from jax.experimental import pallas as pl


def kernel(xyz1, xyz2, color1, color2, params):
    raise NotImplementedError("write your pallas kernel here")


# speedup vs baseline: 4.9670x; 4.9670x over previous
import functools

import jax
import jax.numpy as jnp
from jax import lax
from jax.experimental import pallas as pl
from jax.experimental.pallas import tpu as pltpu

FLOW_NEI = 32
FEAT_NEI = 16
N0 = 8192
_SIZES = (2048, 512, 128, 64)

_INF = 3e38
_BIGI = 2**30


def _pcall(*args, **kwargs):
    return pl.pallas_call(*args, **kwargs)


def _relu(x):
    return jnp.maximum(x, 0.0)


def _mm(a, w):
    return jnp.dot(a, w, preferred_element_type=jnp.float32)


def _fps_body(x_ref, y_ref, z_ref, out_ref, *, npoint, m):
    xs = x_ref[0]
    ys = y_ref[0]
    zs = z_ref[0]
    fio = (lax.broadcasted_iota(jnp.int32, (8, m), 0) * m
           + lax.broadcasted_iota(jnp.int32, (8, m), 1))

    def pick(arr, nx):
        return jnp.sum(jnp.where(fio == nx, arr, 0.0))

    out_ref[0, 0:1, :] = jnp.zeros((1, 1), jnp.int32)
    z0 = jnp.int32(0)
    st0 = (jnp.full((8, m), 1e10, jnp.float32),
           pick(xs, z0), pick(ys, z0), pick(zs, z0))

    def body(i, st):
        d, lx, ly, lz = st
        dn = (xs - lx) ** 2 + (ys - ly) ** 2 + (zs - lz) ** 2
        d = jnp.minimum(d, dn)
        mx = jnp.max(d)
        nx = jnp.min(jnp.where(d == mx, fio, _BIGI))
        out_ref[0, pl.ds(i, 1), :] = jnp.broadcast_to(nx, (1, 1))
        return d, pick(xs, nx), pick(ys, nx), pick(zs, nx)

    lax.fori_loop(1, npoint, body, st0)


def _fps_pair(pc2, npoint):
    n = pc2.shape[1]
    m = n // 8
    xyz = [pc2[:, :, c].reshape(2, 8, m) for c in range(3)]
    out = _pcall(
        functools.partial(_fps_body, npoint=npoint, m=m),
        grid=(2,),
        in_specs=[pl.BlockSpec((1, 8, m), lambda c: (c, 0, 0))] * 3,
        out_specs=pl.BlockSpec((1, npoint, 1), lambda c: (c, 0, 0)),
        out_shape=jax.ShapeDtypeStruct((2, npoint, 1), jnp.int32),
        compiler_params=pltpu.CompilerParams(
            dimension_semantics=("parallel",)),
    )(*xyz)
    return out[:, :, 0]


def _knn_body(q_ref, pt_ref, d2_ref, idx_ref, *, k, np_, ch):
    qb = q_ref[...]
    qq = jnp.sum(qb * qb, axis=1, keepdims=True)
    nch = np_ // ch
    cvals, cidxs = [], []
    for c in range(nch):
        ptc = pt_ref[:, c * ch:(c + 1) * ch]
        ppc = jnp.sum(ptc * ptc, axis=0, keepdims=True)
        d2c = qq + ppc - 2.0 * _mm(qb, ptc)
        lio = lax.broadcasted_iota(jnp.int32, d2c.shape, 1)
        for _ in range(k):
            mv = jnp.min(d2c, axis=1, keepdims=True)
            pos = jnp.min(jnp.where(d2c == mv, lio, _BIGI), axis=1,
                          keepdims=True)
            cvals.append(mv)
            cidxs.append(pos + c * ch)
            d2c = jnp.where(lio == pos, _INF, d2c)
    if nch == 1:
        d2_ref[...] = jnp.concatenate(cvals, axis=1)
        idx_ref[...] = jnp.concatenate(cidxs, axis=1)
        return
    cv = jnp.concatenate(cvals, axis=1)
    ci = jnp.concatenate(cidxs, axis=1)
    ovals, oidxs = [], []
    for _ in range(k):
        mv = jnp.min(cv, axis=1, keepdims=True)
        gi = jnp.min(jnp.where(cv == mv, ci, _BIGI), axis=1, keepdims=True)
        ovals.append(mv)
        oidxs.append(gi)
        cv = jnp.where((cv == mv) & (ci == gi), _INF, cv)
    d2_ref[...] = jnp.concatenate(ovals, axis=1)
    idx_ref[...] = jnp.concatenate(oidxs, axis=1)


def _knn(k, q, p):
    mq = q.shape[0]
    np_ = p.shape[0]
    mb = min(128, mq)
    ch = min(512, np_)
    pt = p.T
    d2, idx = _pcall(
        functools.partial(_knn_body, k=k, np_=np_, ch=ch),
        grid=(mq // mb,),
        in_specs=[pl.BlockSpec((mb, 3), lambda i: (i, 0)),
                  pl.BlockSpec((3, np_), lambda i: (0, 0))],
        out_specs=[pl.BlockSpec((mb, k), lambda i: (i, 0)),
                   pl.BlockSpec((mb, k), lambda i: (i, 0))],
        out_shape=[jax.ShapeDtypeStruct((mq, k), jnp.float32),
                   jax.ShapeDtypeStruct((mq, k), jnp.int32)],
        compiler_params=pltpu.CompilerParams(
            dimension_semantics=("parallel",)),
    )(q, pt)
    return d2, idx


def _mlp0_body(x_ref, w1, b1, w2, b2, w3, b3, out_ref):
    h = _relu(_mm(x_ref[...], w1[...]) + b1[...])
    h = _relu(_mm(h, w2[...]) + b2[...])
    out_ref[...] = _relu(_mm(h, w3[...]) + b3[...])


def _mlp0(layers, x):
    n = x.shape[0]
    mb = 1024
    args = []
    for lp in layers:
        args += [lp['W'], lp['b'][None, :]]
    wspecs = [pl.BlockSpec(a.shape, lambda i: (0, 0)) for a in args]
    cout = layers[-1]['W'].shape[1]
    return _pcall(
        _mlp0_body,
        grid=(n // mb,),
        in_specs=[pl.BlockSpec((mb, 3), lambda i: (i, 0))] + wspecs,
        out_specs=pl.BlockSpec((mb, cout), lambda i: (i, 0)),
        out_shape=jax.ShapeDtypeStruct((n, cout), jnp.float32),
        compiler_params=pltpu.CompilerParams(
            dimension_semantics=("parallel",)),
    )(x, *args)


def _down_body(gf_ref, gp_ref, npc_ref, wv, bv, wp, bp, wa, ba, out_ref, *, k):
    wv_ = wv[...]
    bv_ = bv[...]
    wp_ = wp[...]
    bp_ = bp[...]
    wa_ = wa[...]
    ba_ = ba[...]
    npcb = npc_ref[...]

    def vl(kk):
        v = (_mm(gf_ref[:, kk, :], wv_) + bv_
             + _mm(gp_ref[:, kk, :] - npcb, wp_) + bp_)
        return v, _mm(_relu(v), wa_) + ba_

    mx = None
    for kk in range(k):
        _, l = vl(kk)
        mx = l if mx is None else jnp.maximum(mx, l)
    s = jnp.zeros_like(mx)
    acc = jnp.zeros_like(mx)
    for kk in range(k):
        v, l = vl(kk)
        e = jnp.exp(l - mx)
        s = s + e
        acc = acc + e * v
    out_ref[...] = acc / s


def _down_attn(p, gf, gp, npc, k):
    m, _, cin = gf.shape
    cout = p['v']['W'].shape[1]
    mb = min(128, m)
    args = [p['v']['W'], p['v']['b'][None, :], p['pos']['W'],
            p['pos']['b'][None, :], p['attn']['W'], p['attn']['b'][None, :]]
    wspecs = [pl.BlockSpec(a.shape, lambda i: (0, 0)) for a in args]
    return _pcall(
        functools.partial(_down_body, k=k),
        grid=(m // mb,),
        in_specs=[pl.BlockSpec((mb, k, cin), lambda i: (i, 0, 0)),
                  pl.BlockSpec((mb, k, 3), lambda i: (i, 0, 0)),
                  pl.BlockSpec((mb, 3), lambda i: (i, 0))] + wspecs,
        out_specs=pl.BlockSpec((mb, cout), lambda i: (i, 0)),
        out_shape=jax.ShapeDtypeStruct((m, cout), jnp.float32),
        compiler_params=pltpu.CompilerParams(
            dimension_semantics=("parallel",)),
    )(gf, gp, npc, *args)


def _up_body(ff_ref, gfc_ref, d2_ref, wp1, wp2, bp, out_ref, *, k):
    nd = -d2_ref[...]
    mx = jnp.max(nd, axis=1, keepdims=True)
    e = jnp.exp(nd - mx)
    w = e / jnp.sum(e, axis=1, keepdims=True)
    cc = gfc_ref.shape[2]
    acc = jnp.zeros((ff_ref.shape[0], cc), jnp.float32)
    for kk in range(k):
        acc = acc + gfc_ref[:, kk, :] * w[:, kk:kk + 1]
    out_ref[...] = _relu(_mm(ff_ref[...], wp1[...]) + _mm(acc, wp2[...])
                         + bp[...])


def _up(p, ff, gfc, d2, k):
    m, cf = ff.shape
    cc = gfc.shape[2]
    w = p['proj']['W']
    cout = w.shape[1]
    mb = min(128, m)
    args = [w[:cf], w[cf:], p['proj']['b'][None, :]]
    wspecs = [pl.BlockSpec(a.shape, lambda i: (0, 0)) for a in args]
    return _pcall(
        functools.partial(_up_body, k=k),
        grid=(m // mb,),
        in_specs=[pl.BlockSpec((mb, cf), lambda i: (i, 0)),
                  pl.BlockSpec((mb, k, cc), lambda i: (i, 0, 0)),
                  pl.BlockSpec((mb, k), lambda i: (i, 0))] + wspecs,
        out_specs=pl.BlockSpec((mb, cout), lambda i: (i, 0)),
        out_shape=jax.ShapeDtypeStruct((m, cout), jnp.float32),
        compiler_params=pltpu.CompilerParams(
            dimension_semantics=("parallel",)),
    )(ff, gfc, d2, *args)


def _interp3_body(pc_ref, d2_ref, gfl_ref, gco_ref, pw_ref, fup_ref, cup_ref):
    d2 = d2_ref[...]
    w = 1.0 / (d2 + 1e-8)
    w = w / jnp.sum(w, axis=1, keepdims=True)
    m = pc_ref.shape[0]
    fup = jnp.zeros((m, 3), jnp.float32)
    cup = jnp.zeros((m, gco_ref.shape[2]), jnp.float32)
    for kk in range(3):
        wk = w[:, kk:kk + 1]
        fup = fup + gfl_ref[:, kk, :] * wk
        cup = cup + gco_ref[:, kk, :] * wk
    fup_ref[...] = fup
    cup_ref[...] = cup
    pw_ref[...] = pc_ref[...] + fup


def _interp3(pcf, d2, gflow, gcost):
    m = pcf.shape[0]
    cc = gcost.shape[2]
    mb = min(512, m)
    return _pcall(
        _interp3_body,
        grid=(m // mb,),
        in_specs=[pl.BlockSpec((mb, 3), lambda i: (i, 0)),
                  pl.BlockSpec((mb, 3), lambda i: (i, 0)),
                  pl.BlockSpec((mb, 3, 3), lambda i: (i, 0, 0)),
                  pl.BlockSpec((mb, 3, cc), lambda i: (i, 0, 0))],
        out_specs=[pl.BlockSpec((mb, 3), lambda i: (i, 0)),
                   pl.BlockSpec((mb, 3), lambda i: (i, 0)),
                   pl.BlockSpec((mb, cc), lambda i: (i, 0))],
        out_shape=[jax.ShapeDtypeStruct((m, 3), jnp.float32),
                   jax.ShapeDtypeStruct((m, 3), jnp.float32),
                   jax.ShapeDtypeStruct((m, cc), jnp.float32)],
        compiler_params=pltpu.CompilerParams(
            dimension_semantics=("parallel",)),
    )(pcf, d2, gflow, gcost)


def _dec_body(f1_ref, g2_ref, gpt_ref, pw_ref, *rest, k, has_prev):
    if has_prev:
        (cup_ref, fup_ref, wc1, wc2, wc3, bc, wf1, wf2, wf3, bf, w2, b2,
         cost_ref, flow_ref) = rest
    else:
        (wc1, wc2, wc3, bc, wf1, wf2, bf, w2, b2,
         cost_ref, flow_ref) = rest
    f1b = f1_ref[...]
    pwb = pw_ref[...]
    base = _mm(f1b, wc1[...]) + bc[...]
    wc2_ = wc2[...]
    wc3_ = wc3[...]
    cost = None
    for kk in range(k):
        h = _relu(base + _mm(g2_ref[:, kk, :], wc2_)
                  + _mm(gpt_ref[:, kk, :] - pwb, wc3_))
        cost = h if cost is None else jnp.maximum(cost, h)
    hf = _mm(f1b, wf1[...]) + _mm(cost, wf2[...]) + bf[...]
    if has_prev:
        hf = hf + _mm(cup_ref[...], wf3[...])
    hf = _relu(hf)
    fl = _mm(hf, w2[...]) + b2[...]
    if has_prev:
        fl = fl + fup_ref[...]
    cost_ref[...] = cost
    flow_ref[...] = fl


def _decoder(p, f1, g2, gpt, pw, cup, fup, k):
    m, cf = f1.shape
    cc = p['cost']['W'].shape[1]
    has_prev = cup is not None
    wc = p['cost']['W']
    wf = p['flow1']['W']
    mb = min(128, m)
    args = [wc[:cf], wc[cf:2 * cf], wc[2 * cf:], p['cost']['b'][None, :],
            wf[:cf], wf[cf:cf + cc]]
    if has_prev:
        args.append(wf[cf + cc:])
    args += [p['flow1']['b'][None, :], p['flow2']['W'],
             p['flow2']['b'][None, :]]
    ins = [f1, g2, gpt, pw]
    in_specs = [pl.BlockSpec((mb, cf), lambda i: (i, 0)),
                pl.BlockSpec((mb, k, g2.shape[2]), lambda i: (i, 0, 0)),
                pl.BlockSpec((mb, k, 3), lambda i: (i, 0, 0)),
                pl.BlockSpec((mb, 3), lambda i: (i, 0))]
    if has_prev:
        ins += [cup, fup]
        in_specs += [pl.BlockSpec((mb, cup.shape[1]), lambda i: (i, 0)),
                     pl.BlockSpec((mb, 3), lambda i: (i, 0))]
    in_specs += [pl.BlockSpec(a.shape, lambda i: (0, 0)) for a in args]
    cost, flow = _pcall(
        functools.partial(_dec_body, k=k, has_prev=has_prev),
        grid=(m // mb,),
        in_specs=in_specs,
        out_specs=[pl.BlockSpec((mb, cc), lambda i: (i, 0)),
                   pl.BlockSpec((mb, 3), lambda i: (i, 0))],
        out_shape=[jax.ShapeDtypeStruct((m, cc), jnp.float32),
                   jax.ShapeDtypeStruct((m, 3), jnp.float32)],
        compiler_params=pltpu.CompilerParams(
            dimension_semantics=("parallel",)),
    )(*ins, *args)
    return cost, flow


def _take(tab, idx):
    return jnp.take(tab, idx, axis=0)


def _down_level(p, pcs, feats, npoint, k):
    fi2 = _fps_pair(jnp.stack(pcs, axis=0), npoint)
    outs = []
    for c in range(2):
        pc, feat = pcs[c], feats[c]
        npc = _take(pc, fi2[c])
        _, ki = _knn(k, npc, pc)
        gf = _take(feat, ki.reshape(-1)).reshape(npoint, k, feat.shape[1])
        gp = _take(pc, ki.reshape(-1)).reshape(npoint, k, 3)
        outs.append((npc, _down_attn(p, gf, gp, npc, k)))
    return outs


def _up_level(p, pcf, pcc, ff, fc, k):
    d2, idx = _knn(k, pcf, pcc)
    gfc = _take(fc, idx.reshape(-1)).reshape(pcf.shape[0], k, fc.shape[1])
    return _up(p, ff, gfc, d2, k)


def _dec_level(p, pc1, pc2, pc1c, f1, f2, costp, flowp, k):
    m = pc1.shape[0]
    if flowp is None:
        pw, cup, fup = pc1, None, None
    else:
        d2i, i3 = _knn(3, pc1, pc1c)
        gfl = _take(flowp, i3.reshape(-1)).reshape(m, 3, 3)
        gco = _take(costp, i3.reshape(-1)).reshape(m, 3, costp.shape[1])
        pw, fup, cup = _interp3(pc1, d2i, gfl, gco)
    _, idx = _knn(k, pw, pc2)
    g2 = _take(f2, idx.reshape(-1)).reshape(m, k, f2.shape[1])
    gpt = _take(pc2, idx.reshape(-1)).reshape(m, k, 3)
    return _decoder(p, f1, g2, gpt, pw, cup, fup, k)


def kernel(xyz1, xyz2, color1, color2, params):
    p = params
    pc1_0, pc2_0 = xyz1[0], xyz2[0]
    f0 = _mlp0(p['level0'], jnp.concatenate([color1[0], color2[0]], axis=0))
    f1_0, f2_0 = f0[:N0], f0[N0:]

    n1, n2, n3, n4 = _SIZES
    (pc1_1, f1_1), (pc2_1, f2_1) = _down_level(
        p['level1'], (pc1_0, pc2_0), (f1_0, f2_0), n1, FEAT_NEI)
    (pc1_2, f1_2), (pc2_2, f2_2) = _down_level(
        p['level2'], (pc1_1, pc2_1), (f1_1, f2_1), n2, FEAT_NEI)
    (pc1_3, f1_3), (pc2_3, f2_3) = _down_level(
        p['level3'], (pc1_2, pc2_2), (f1_2, f2_2), n3, FEAT_NEI)
    (pc1_4, f1_4), (pc2_4, f2_4) = _down_level(
        p['level4'], (pc1_3, pc2_3), (f1_3, f2_3), n4, FEAT_NEI)

    f1_3u = _up_level(p['up4'], pc1_3, pc1_4, f1_3, f1_4, FEAT_NEI)
    f2_3u = _up_level(p['up4'], pc2_3, pc2_4, f2_3, f2_4, FEAT_NEI)
    f1_2u = _up_level(p['up3'], pc1_2, pc1_3, f1_2, f1_3u, FEAT_NEI)
    f2_2u = _up_level(p['up3'], pc2_2, pc2_3, f2_2, f2_3u, FEAT_NEI)
    f1_1u = _up_level(p['up2'], pc1_1, pc1_2, f1_1, f1_2u, FEAT_NEI)
    f2_1u = _up_level(p['up2'], pc2_1, pc2_2, f2_1, f2_2u, FEAT_NEI)
    f1_0u = _up_level(p['up1'], pc1_0, pc1_1, f1_0, f1_1u, FEAT_NEI)
    f2_0u = _up_level(p['up1'], pc2_0, pc2_1, f2_0, f2_1u, FEAT_NEI)

    c3, fl3 = _dec_level(p['flow3'], pc1_3, pc2_3, pc1_4, f1_3u, f2_3u,
                         None, None, FLOW_NEI)
    c2, fl2 = _dec_level(p['flow2'], pc1_2, pc2_2, pc1_3, f1_2u, f2_2u,
                         c3, fl3, FLOW_NEI)
    c1, fl1 = _dec_level(p['flow1'], pc1_1, pc2_1, pc1_2, f1_1u, f2_1u,
                         c2, fl2, FLOW_NEI)
    _, fl0 = _dec_level(p['flow0'], pc1_0, pc2_0, pc1_1, f1_0u, f2_0u,
                        c1, fl1, FLOW_NEI)

    return fl0[None], fl1[None], fl2[None], fl3[None]
